# baseline (speedup 1.0000x reference)
import functools

import jax
import jax.numpy as jnp
from jax import lax
from jax.experimental import pallas as pl
from jax.experimental.pallas import tpu as pltpu
from jax.experimental.pallas import tpu_sc as plsc

BATCH = 16384
EDIM = 64
BLK = 128

NUM_CORES = 2
NUM_SUBCORES = 16
NUM_WORKERS = NUM_CORES * NUM_SUBCORES
SC_BATCH = 6656
BPW = SC_BATCH // NUM_WORKERS
LANES = 16
NSLOT = 4

TC_BATCH = BATCH - SC_BATCH
TC_G = 8
TC_STEPS = TC_BATCH // TC_G


def _sc_body(user_wT, item_wT, user_idx, item_idx, out,
             idx_us, idx_is, idx_v, blk_u, blk_i, out_v, sems):
    wid = lax.axis_index("s") * NUM_CORES + lax.axis_index("c")
    base = wid * BPW

    pltpu.sync_copy(user_idx.at[pl.ds(base, BPW)], idx_v)

    def unpack_u(g, carry):
        v = idx_v[pl.ds(g * LANES, LANES)]
        for l in range(LANES):
            idx_us[g * LANES + l] = v[l]
        return carry

    lax.fori_loop(0, BPW // LANES, unpack_u, 0)

    pltpu.sync_copy(item_idx.at[pl.ds(base, BPW)], idx_v)

    def unpack_i(g, carry):
        v = idx_v[pl.ds(g * LANES, LANES)]
        for l in range(LANES):
            idx_is[g * LANES + l] = v[l]
        return carry

    lax.fori_loop(0, BPW // LANES, unpack_i, 0)

    lane_iota = lax.iota(jnp.int32, LANES)

    def block_start(idx):
        return pl.multiple_of((idx >> 7) * BLK, BLK)

    def fire(j, slot):
        bu = block_start(idx_us[j])
        bi = block_start(idx_is[j])
        pltpu.async_copy(
            user_wT.at[:, pl.ds(bu, BLK)], blk_u.at[slot], sems.at[slot])
        pltpu.async_copy(
            item_wT.at[:, pl.ds(bi, BLK)], blk_i.at[slot], sems.at[slot])

    def consume(j, slot, acc):
        iu = idx_us[j]
        ii = idx_is[j]
        bu = block_start(iu)
        bi = block_start(ii)
        pltpu.make_async_copy(
            user_wT.at[:, pl.ds(bu, BLK)], blk_u.at[slot],
            sems.at[slot]).wait()
        pltpu.make_async_copy(
            item_wT.at[:, pl.ds(bi, BLK)], blk_i.at[slot],
            sems.at[slot]).wait()
        cu = jnp.full((LANES,), iu & (BLK - 1), jnp.int32)
        ci = jnp.full((LANES,), ii & (BLK - 1), jnp.int32)
        slot_v = jnp.full((LANES,), slot, jnp.int32)
        p = jnp.zeros((LANES,), jnp.float32)
        for k in range(EDIM // LANES):
            rows = lane_iota + (k * LANES)
            gu = plsc.load_gather(blk_u, [slot_v, rows, cu])
            gi = plsc.load_gather(blk_i, [slot_v, rows, ci])
            p = p + gu * gi
        s = lax.reduce_sum(p, axes=(0,))
        acc = jnp.where(lane_iota == (j & (LANES - 1)), s, acc)

        @pl.when((j & (LANES - 1)) == (LANES - 1))
        def _():
            out_v[pl.ds((j >> 4) * LANES, LANES)] = acc

        return acc

    for j in range(NSLOT):
        fire(j, j)

    def body(j, acc):
        slot = lax.rem(j, NSLOT)
        acc = consume(j, slot, acc)
        fire(j + NSLOT, slot)
        return acc

    acc = lax.fori_loop(0, BPW - NSLOT, body, jnp.zeros((LANES,), jnp.float32))

    def tail(j, acc):
        return consume(j, lax.rem(j, NSLOT), acc)

    lax.fori_loop(BPW - NSLOT, BPW, tail, acc)

    pltpu.sync_copy(out_v, out.at[pl.ds(base, BPW)])


_sc_kernel = functools.partial(
    pl.kernel,
    out_type=jax.ShapeDtypeStruct((SC_BATCH,), jnp.float32),
    mesh=plsc.VectorSubcoreMesh(core_axis_name="c", subcore_axis_name="s"),
    scratch_types=[
        pltpu.SMEM((BPW,), jnp.int32),
        pltpu.SMEM((BPW,), jnp.int32),
        pltpu.VMEM((BPW,), jnp.int32),
        pltpu.VMEM((NSLOT, EDIM, BLK), jnp.float32),
        pltpu.VMEM((NSLOT, EDIM, BLK), jnp.float32),
        pltpu.VMEM((BPW,), jnp.float32),
        pltpu.SemaphoreType.DMA((NSLOT,)),
    ],
    compiler_params=pltpu.CompilerParams(needs_layout_passes=False),
)(_sc_body)


def _tc_body(uidx, iidx, *refs):
    (*blk_refs, out_ref) = refs
    u_refs = blk_refs[:TC_G]
    i_refs = blk_refs[TC_G:]
    step = pl.program_id(0)
    colid = lax.broadcasted_iota(jnp.int32, (EDIM, BLK), 1)
    scores = []
    for l in range(TC_G):
        cu = uidx[step * TC_G + l] & (BLK - 1)
        ci = iidx[step * TC_G + l] & (BLK - 1)
        ucol = jnp.where(colid == cu, u_refs[l][...], 0.0)
        icol = jnp.where(colid == ci, i_refs[l][...], 0.0)
        u = jnp.sum(ucol, axis=1)
        v = jnp.sum(icol, axis=1)
        scores.append(jnp.sum(u * v))
    out_ref[pl.ds(step, 1), :] = jnp.stack(scores).reshape(1, TC_G)


def _tc_gather_dot(user_wT, item_wT, uidx, iidx):
    def u_map(l):
        return lambda s, uu, ii: (0, uu[s * TC_G + l] >> 7)

    def i_map(l):
        return lambda s, uu, ii: (0, ii[s * TC_G + l] >> 7)

    in_specs = (
        [pl.BlockSpec((EDIM, BLK), u_map(l)) for l in range(TC_G)]
        + [pl.BlockSpec((EDIM, BLK), i_map(l)) for l in range(TC_G)]
    )
    grid_spec = pltpu.PrefetchScalarGridSpec(
        num_scalar_prefetch=2,
        grid=(TC_STEPS,),
        in_specs=in_specs,
        out_specs=pl.BlockSpec((TC_STEPS, TC_G), lambda s, uu, ii: (0, 0)),
    )
    out = pl.pallas_call(
        _tc_body,
        grid_spec=grid_spec,
        out_shape=jax.ShapeDtypeStruct((TC_STEPS, TC_G), jnp.float32),
    )(uidx, iidx, *([user_wT] * TC_G), *([item_wT] * TC_G))
    return out.reshape(TC_BATCH)


def kernel(user_embs_weight, item_embs_weight, user, item):
    user = user.astype(jnp.int32)
    item = item.astype(jnp.int32)
    user_wT = user_embs_weight.T
    item_wT = item_embs_weight.T
    sc_out = _sc_kernel(user_wT, item_wT, user[:SC_BATCH], item[:SC_BATCH])
    tc_out = _tc_gather_dot(user_wT, item_wT, user[SC_BATCH:], item[SC_BATCH:])
    return jnp.concatenate([sc_out, tc_out])


# speedup vs baseline: 3.5449x; 3.5449x over previous
import functools

import jax
import jax.numpy as jnp
from jax import lax
from jax.experimental import pallas as pl
from jax.experimental.pallas import tpu as pltpu
from jax.experimental.pallas import tpu_sc as plsc

BATCH = 16384
EDIM = 64
NUM_CORES = 2
NUM_SUBCORES = 16
NUM_WORKERS = NUM_CORES * NUM_SUBCORES
BPW = BATCH // NUM_WORKERS
LANES = 16
BLK = 128
NSLOT = 6


def _sc_body(user_wT, item_wT, user_idx, item_idx, out,
             idx_us, idx_is, idx_v, blk_u, blk_i, out_v, sems):
    wid = lax.axis_index("s") * NUM_CORES + lax.axis_index("c")
    base = wid * BPW

    pltpu.sync_copy(user_idx.at[pl.ds(base, BPW)], idx_v)

    def unpack_u(g, carry):
        v = idx_v[pl.ds(g * LANES, LANES)]
        for l in range(LANES):
            idx_us[g * LANES + l] = v[l]
        return carry

    lax.fori_loop(0, BPW // LANES, unpack_u, 0)

    pltpu.sync_copy(item_idx.at[pl.ds(base, BPW)], idx_v)

    def unpack_i(g, carry):
        v = idx_v[pl.ds(g * LANES, LANES)]
        for l in range(LANES):
            idx_is[g * LANES + l] = v[l]
        return carry

    lax.fori_loop(0, BPW // LANES, unpack_i, 0)

    lane_iota = lax.iota(jnp.int32, LANES)

    def block_start(idx):
        return pl.multiple_of((idx >> 7) * BLK, BLK)

    def fire(j, slot):
        bu = block_start(idx_us[j])
        bi = block_start(idx_is[j])
        pltpu.async_copy(
            user_wT.at[:, pl.ds(bu, BLK)], blk_u.at[slot], sems.at[slot])
        pltpu.async_copy(
            item_wT.at[:, pl.ds(bi, BLK)], blk_i.at[slot], sems.at[slot])

    def consume(j, slot, acc):
        iu = idx_us[j]
        ii = idx_is[j]
        bu = block_start(iu)
        bi = block_start(ii)
        pltpu.make_async_copy(
            user_wT.at[:, pl.ds(bu, BLK)], blk_u.at[slot],
            sems.at[slot]).wait()
        pltpu.make_async_copy(
            item_wT.at[:, pl.ds(bi, BLK)], blk_i.at[slot],
            sems.at[slot]).wait()
        cu = jnp.full((LANES,), iu & (BLK - 1), jnp.int32)
        ci = jnp.full((LANES,), ii & (BLK - 1), jnp.int32)
        slot_v = jnp.full((LANES,), slot, jnp.int32)
        p = jnp.zeros((LANES,), jnp.float32)
        for k in range(EDIM // LANES):
            rows = lane_iota + (k * LANES)
            gu = plsc.load_gather(blk_u, [slot_v, rows, cu])
            gi = plsc.load_gather(blk_i, [slot_v, rows, ci])
            p = p + gu * gi
        s = lax.reduce_sum(p, axes=(0,))
        acc = jnp.where(lane_iota == (j & (LANES - 1)), s, acc)

        @pl.when((j & (LANES - 1)) == (LANES - 1))
        def _():
            out_v[pl.ds((j >> 4) * LANES, LANES)] = acc

        return acc

    for j in range(NSLOT):
        fire(j, j)

    def body(j, acc):
        slot = lax.rem(j, NSLOT)
        acc = consume(j, slot, acc)
        fire(j + NSLOT, slot)
        return acc

    acc = lax.fori_loop(0, BPW - NSLOT, body, jnp.zeros((LANES,), jnp.float32))

    def tail(j, acc):
        return consume(j, lax.rem(j, NSLOT), acc)

    lax.fori_loop(BPW - NSLOT, BPW, tail, acc)

    pltpu.sync_copy(out_v, out.at[pl.ds(base, BPW)])


_sc_kernel = functools.partial(
    pl.kernel,
    out_type=jax.ShapeDtypeStruct((BATCH,), jnp.float32),
    mesh=plsc.VectorSubcoreMesh(core_axis_name="c", subcore_axis_name="s"),
    scratch_types=[
        pltpu.SMEM((BPW,), jnp.int32),
        pltpu.SMEM((BPW,), jnp.int32),
        pltpu.VMEM((BPW,), jnp.int32),
        pltpu.VMEM((NSLOT, EDIM, BLK), jnp.float32),
        pltpu.VMEM((NSLOT, EDIM, BLK), jnp.float32),
        pltpu.VMEM((BPW,), jnp.float32),
        pltpu.SemaphoreType.DMA((NSLOT,)),
    ],
    compiler_params=pltpu.CompilerParams(needs_layout_passes=False),
)(_sc_body)


def kernel(user_embs_weight, item_embs_weight, user, item):
    user = user.astype(jnp.int32)
    item = item.astype(jnp.int32)
    return _sc_kernel(user_embs_weight.T, item_embs_weight.T, user, item)
